# baseline (speedup 1.0000x reference)
import functools

import jax
import jax.numpy as jnp
from jax import lax
from jax.experimental import pallas as pl
from jax.experimental.pallas import tpu as pltpu
from jax.experimental.pallas import tpu_sc as plsc

N_TABLES = 26
SPLITS = (18, 8)
DIM = 32
N_OUT = 5
NC, NS = 2, 16
NW = NC * NS
VPAD = 100352
CHUNK = 8192


def _proj_body(w_ref, t_ref, out_ref):
    res = jax.lax.dot_general(
        w_ref[...],
        t_ref[0],
        (((0,), (0,)), ((), ())),
        preferred_element_type=jnp.float32,
    )
    out_ref[0] = res.reshape(N_OUT, res.shape[1] // 128, 128)


def _tc_project(w8, tables_t, j0, nj):
    dim = tables_t.shape[1]
    blk = VPAD // 2
    kb = blk // 128
    return pl.pallas_call(
        _proj_body,
        grid=(nj, 2),
        in_specs=[
            pl.BlockSpec((DIM, N_OUT), lambda j, c: (j0 + j, 0)),
            pl.BlockSpec((1, dim, blk), lambda j, c: (j0 + j, 0, c)),
        ],
        out_specs=pl.BlockSpec((1, N_OUT, kb, 128), lambda j, c: (j, 0, c, 0)),
        out_shape=jax.ShapeDtypeStruct((nj, N_OUT, VPAD // 128, 128), jnp.float32),
    )(w8, tables_t)


def _lookup_body(j0, nj, idx_hbm, p_hbm, val_hbm, row_v, idx_v, val_v, sem):
    wid = lax.axis_index("s") * NC + lax.axis_index("c")
    batch = idx_hbm.shape[1]
    n_chunks = batch // CHUNK
    n_tasks = nj * N_OUT
    rounds = -(-n_tasks // NW)

    @pl.loop(0, rounds)
    def _task_loop(s):
        t = s * NW + wid

        @pl.when(t < n_tasks)
        def _():
            j = t // N_OUT
            o = lax.rem(t, N_OUT)
            pltpu.sync_copy(p_hbm.at[j, o], row_v)

            @pl.loop(0, n_chunks)
            def _chunk(c):
                pltpu.sync_copy(
                    idx_hbm.at[j0 + j, pl.ds(c * CHUNK, CHUNK)], idx_v
                )

                @pl.loop(0, CHUNK // 16, unroll=8)
                def _group(g):
                    iv = idx_v[pl.ds(g * 16, 16)]
                    val_v[g >> 3, pl.ds((g & 7) * 16, 16)] = plsc.load_gather(
                        row_v, [iv >> 7, iv & 127]
                    )

                pltpu.sync_copy(
                    val_v,
                    val_hbm.at[j, o, pl.ds(c * (CHUNK // 128), CHUNK // 128), :],
                )


def _sc_lookup(inputs, p, j0, nj):
    batch = inputs.shape[1]
    mesh = plsc.VectorSubcoreMesh(core_axis_name="c", subcore_axis_name="s")
    return pl.kernel(
        functools.partial(_lookup_body, j0, nj),
        out_type=jax.ShapeDtypeStruct((nj, N_OUT, batch // 128, 128), jnp.float32),
        mesh=mesh,
        scratch_types=[
            pltpu.VMEM((VPAD // 128, 128), jnp.float32),
            pltpu.VMEM((CHUNK,), jnp.int32),
            pltpu.VMEM((CHUNK // 128, 128), jnp.float32),
            pltpu.SemaphoreType.DMA,
        ],
        compiler_params=pltpu.CompilerParams(
            use_tc_tiling_on_sc=False, needs_layout_passes=False
        ),
    )(inputs, p)


def _reduce_body(*refs):
    val_refs, b_ref, out_ref = refs[:-2], refs[-2], refs[-1]
    nrow = val_refs[0].shape[2]
    acc = jnp.zeros((N_OUT, nrow, 128), dtype=jnp.float32)
    for vr in val_refs:
        for j in range(vr.shape[0]):
            acc = acc + vr[j]
    out_ref[...] = acc.reshape(N_OUT, nrow * 128) + b_ref[...]


def _tc_reduce(vals, b2d):
    batch = vals[0].shape[2] * 128
    blk = 4096
    in_specs = [
        pl.BlockSpec((v.shape[0], N_OUT, blk // 128, 128), lambda i: (0, 0, i, 0))
        for v in vals
    ] + [pl.BlockSpec((N_OUT, 1), lambda i: (0, 0))]
    return pl.pallas_call(
        _reduce_body,
        grid=(batch // blk,),
        in_specs=in_specs,
        out_specs=pl.BlockSpec((N_OUT, blk), lambda i: (0, i)),
        out_shape=jax.ShapeDtypeStruct((N_OUT, batch), jnp.float32),
    )(*vals, b2d)


@jax.jit
def kernel(inputs, tables, W, b):
    n, vocab, dim = tables.shape
    tables_t = jnp.transpose(tables, (0, 2, 1))
    vals = []
    j0 = 0
    for nj in SPLITS:
        p_g = _tc_project(W, tables_t, j0, nj)
        vals.append(_sc_lookup(inputs, p_g, j0, nj))
        j0 += nj
    out5 = _tc_reduce(vals, b.reshape(-1, 1))
    return out5.T


# speedup vs baseline: 1.0099x; 1.0099x over previous
import functools

import jax
import jax.numpy as jnp
from jax import lax
from jax.experimental import pallas as pl
from jax.experimental.pallas import tpu as pltpu
from jax.experimental.pallas import tpu_sc as plsc

N_TABLES = 26
SPLITS = (17, 9)
DIM = 32
N_OUT = 5
NC, NS = 2, 16
NW = NC * NS
VPAD = 100352
CHUNK = 8192


def _proj_body(w_ref, t_ref, out_ref):
    res = jax.lax.dot_general(
        w_ref[...],
        t_ref[0],
        (((0,), (0,)), ((), ())),
        preferred_element_type=jnp.float32,
    )
    out_ref[0] = res.reshape(N_OUT, res.shape[1] // 128, 128)


def _tc_project(w, tables_t, j0, nj):
    dim = tables_t.shape[1]
    blk = VPAD // 2
    kb = blk // 128
    return pl.pallas_call(
        _proj_body,
        grid=(nj, 2),
        in_specs=[
            pl.BlockSpec((DIM, N_OUT), lambda j, c: (j0 + j, 0)),
            pl.BlockSpec((1, dim, blk), lambda j, c: (j0 + j, 0, c)),
        ],
        out_specs=pl.BlockSpec((1, N_OUT, kb, 128), lambda j, c: (j, 0, c, 0)),
        out_shape=jax.ShapeDtypeStruct((nj, N_OUT, VPAD // 128, 128), jnp.float32),
    )(w, tables_t)


def _lookup_body(j0, nj, idx_hbm, p_hbm, val_hbm, row_v, idx_v, val_v, sem):
    wid = lax.axis_index("s") * NC + lax.axis_index("c")
    batch = idx_hbm.shape[1]
    n_chunks = batch // CHUNK
    n_tasks = nj * N_OUT
    rounds = -(-n_tasks // NW)

    @pl.loop(0, rounds)
    def _task_loop(s):
        t = s * NW + wid

        @pl.when(t < n_tasks)
        def _():
            j = t // N_OUT
            o = lax.rem(t, N_OUT)
            pltpu.sync_copy(p_hbm.at[j, o], row_v)

            @pl.loop(0, n_chunks)
            def _chunk(c):
                pltpu.sync_copy(
                    idx_hbm.at[j0 + j, pl.ds(c * CHUNK, CHUNK)], idx_v
                )

                @pl.loop(0, CHUNK // 16, unroll=8)
                def _group(g):
                    iv = idx_v[pl.ds(g * 16, 16)]
                    val_v[g >> 3, pl.ds((g & 7) * 16, 16)] = plsc.load_gather(
                        row_v, [iv >> 7, iv & 127]
                    )

                pltpu.sync_copy(
                    val_v,
                    val_hbm.at[j, o, pl.ds(c * (CHUNK // 128), CHUNK // 128), :],
                )


def _sc_lookup(inputs, p, j0, nj):
    batch = inputs.shape[1]
    mesh = plsc.VectorSubcoreMesh(core_axis_name="c", subcore_axis_name="s")
    return pl.kernel(
        functools.partial(_lookup_body, j0, nj),
        out_type=jax.ShapeDtypeStruct((nj, N_OUT, batch // 128, 128), jnp.float32),
        mesh=mesh,
        scratch_types=[
            pltpu.VMEM((VPAD // 128, 128), jnp.float32),
            pltpu.VMEM((CHUNK,), jnp.int32),
            pltpu.VMEM((CHUNK // 128, 128), jnp.float32),
            pltpu.SemaphoreType.DMA,
        ],
        compiler_params=pltpu.CompilerParams(
            use_tc_tiling_on_sc=False, needs_layout_passes=False
        ),
    )(inputs, p)


def _reduce_body(*refs):
    val_refs, b_ref, out_ref = refs[:-2], refs[-2], refs[-1]
    nrow = val_refs[0].shape[2]
    acc = jnp.zeros((N_OUT, nrow, 128), dtype=jnp.float32)
    for vr in val_refs:
        for j in range(vr.shape[0]):
            acc = acc + vr[j]
    out_ref[...] = acc.reshape(N_OUT, nrow * 128) + b_ref[...]


def _tc_reduce(vals, b2d):
    batch = vals[0].shape[2] * 128
    blk = 4096
    in_specs = [
        pl.BlockSpec((v.shape[0], N_OUT, blk // 128, 128), lambda i: (0, 0, i, 0))
        for v in vals
    ] + [pl.BlockSpec((N_OUT, 1), lambda i: (0, 0))]
    return pl.pallas_call(
        _reduce_body,
        grid=(batch // blk,),
        in_specs=in_specs,
        out_specs=pl.BlockSpec((N_OUT, blk), lambda i: (0, i)),
        out_shape=jax.ShapeDtypeStruct((N_OUT, batch), jnp.float32),
    )(*vals, b2d)


@jax.jit
def kernel(inputs, tables, W, b):
    n, vocab, dim = tables.shape
    tables_t = jnp.transpose(tables, (0, 2, 1))
    vals = []
    j0 = 0
    for nj in SPLITS:
        p_g = _tc_project(W, tables_t, j0, nj)
        vals.append(_sc_lookup(inputs, p_g, j0, nj))
        j0 += nj
    out5 = _tc_reduce(vals, b.reshape(-1, 1))
    return out5.T
